# baseline (speedup 1.0000x reference)
import functools

import jax
import jax.numpy as jnp
from jax import lax
from jax.experimental import pallas as pl
from jax.experimental.pallas import tpu as pltpu
from jax.experimental.pallas import tpu_sc as plsc

_BATCH = 4096
_PIXELS = 12288


def _sc_gather(idx, a_flat, b_flat):
    info = plsc.get_sparse_core_info()
    nw = info.num_cores * info.num_subcores
    b_per_w = _BATCH // nw

    mesh = plsc.VectorSubcoreMesh(core_axis_name="c", subcore_axis_name="s")

    @functools.partial(
        pl.kernel,
        out_type=(
            jax.ShapeDtypeStruct((_BATCH,), jnp.float32),
            jax.ShapeDtypeStruct((_BATCH,), jnp.float32),
        ),
        mesh=mesh,
        scratch_types=[
            pltpu.VMEM((b_per_w,), jnp.int32),
            pltpu.VMEM((b_per_w,), jnp.float32),
            pltpu.VMEM((b_per_w,), jnp.float32),
            pltpu.SemaphoreType.DMA,
            pltpu.SemaphoreType.DMA,
        ],
    )
    def sc_kernel(idx_hbm, a_hbm, b_hbm, scale_hbm, off_hbm,
                  idx_v, a_v, b_v, sem_a, sem_b):
        wid = lax.axis_index("s") * info.num_cores + lax.axis_index("c")
        base = wid * b_per_w
        pltpu.sync_copy(idx_hbm.at[pl.ds(base, b_per_w)], idx_v)
        cp_a = pltpu.async_copy(a_hbm.at[idx_v], a_v, sem_a)
        cp_b = pltpu.async_copy(b_hbm.at[idx_v], b_v, sem_b)
        cp_a.wait()
        cp_b.wait()
        for i in range(b_per_w // 16):
            sl = pl.ds(i * 16, 16)
            a_v[sl] = jnp.exp(a_v[sl])
        pltpu.sync_copy(a_v, scale_hbm.at[pl.ds(base, b_per_w)])
        pltpu.sync_copy(b_v, off_hbm.at[pl.ds(base, b_per_w)])

    return sc_kernel(idx, a_flat, b_flat)


def _tc_apply(scale, offset, image):
    rows = 64
    grid = (_BATCH // rows,)

    def body(s_ref, o_ref, img_ref, out_ref):
        out_ref[...] = jnp.clip(
            s_ref[...] * img_ref[...] + o_ref[...], 0.0, 1.0)

    return pl.pallas_call(
        body,
        grid=grid,
        in_specs=[
            pl.BlockSpec((rows, 1), lambda i: (i, 0)),
            pl.BlockSpec((rows, 1), lambda i: (i, 0)),
            pl.BlockSpec((rows, _PIXELS), lambda i: (i, 0)),
        ],
        out_specs=pl.BlockSpec((rows, _PIXELS), lambda i: (i, 0)),
        out_shape=jax.ShapeDtypeStruct((_BATCH, _PIXELS), jnp.float32),
    )(scale.reshape(_BATCH, 1), offset.reshape(_BATCH, 1), image)


def kernel(idx, image, exposure_a, exposure_b):
    scale, offset = _sc_gather(idx, exposure_a.reshape(-1),
                               exposure_b.reshape(-1))
    return _tc_apply(scale, offset, image)


# speedup vs baseline: 1.0823x; 1.0823x over previous
import functools

import jax
import jax.numpy as jnp
from jax import lax
from jax.experimental import pallas as pl
from jax.experimental.pallas import tpu as pltpu
from jax.experimental.pallas import tpu_sc as plsc

_BATCH = 4096
_PIXELS = 12288


def _sc_gather(idx, a_flat, b_flat):
    info = plsc.get_sparse_core_info()
    nw = info.num_cores * info.num_subcores
    b_per_w = _BATCH // nw

    mesh = plsc.VectorSubcoreMesh(core_axis_name="c", subcore_axis_name="s")

    @functools.partial(
        pl.kernel,
        out_type=(
            jax.ShapeDtypeStruct((_BATCH,), jnp.float32),
            jax.ShapeDtypeStruct((_BATCH,), jnp.float32),
        ),
        mesh=mesh,
        scratch_types=[
            pltpu.VMEM((b_per_w,), jnp.int32),
            pltpu.VMEM((b_per_w,), jnp.float32),
            pltpu.VMEM((b_per_w,), jnp.float32),
            pltpu.SemaphoreType.DMA,
            pltpu.SemaphoreType.DMA,
        ],
    )
    def sc_kernel(idx_hbm, a_hbm, b_hbm, scale_hbm, off_hbm,
                  idx_v, a_v, b_v, sem_a, sem_b):
        wid = lax.axis_index("s") * info.num_cores + lax.axis_index("c")
        base = wid * b_per_w
        pltpu.sync_copy(idx_hbm.at[pl.ds(base, b_per_w)], idx_v)
        cp_a = pltpu.async_copy(a_hbm.at[idx_v], a_v, sem_a)
        cp_b = pltpu.async_copy(b_hbm.at[idx_v], b_v, sem_b)
        cp_a.wait()
        cp_b.wait()
        for i in range(b_per_w // 16):
            sl = pl.ds(i * 16, 16)
            a_v[sl] = jnp.exp(a_v[sl])
        pltpu.sync_copy(a_v, scale_hbm.at[pl.ds(base, b_per_w)])
        pltpu.sync_copy(b_v, off_hbm.at[pl.ds(base, b_per_w)])

    return sc_kernel(idx, a_flat, b_flat)


def _tc_apply(scale, offset, image):
    rows = 128
    grid = (_BATCH // rows,)

    def body(s_ref, o_ref, img_ref, out_ref):
        s = s_ref[...].reshape(rows, 1)
        o = o_ref[...].reshape(rows, 1)
        out_ref[...] = jnp.clip(s * img_ref[...] + o, 0.0, 1.0)

    return pl.pallas_call(
        body,
        grid=grid,
        in_specs=[
            pl.BlockSpec((rows,), lambda i: (i,)),
            pl.BlockSpec((rows,), lambda i: (i,)),
            pl.BlockSpec((rows, _PIXELS), lambda i: (i, 0)),
        ],
        out_specs=pl.BlockSpec((rows, _PIXELS), lambda i: (i, 0)),
        out_shape=jax.ShapeDtypeStruct((_BATCH, _PIXELS), jnp.float32),
    )(scale, offset, image)


def kernel(idx, image, exposure_a, exposure_b):
    scale, offset = _sc_gather(idx, exposure_a.reshape(-1),
                               exposure_b.reshape(-1))
    return _tc_apply(scale, offset, image)


# speedup vs baseline: 1.0921x; 1.0091x over previous
import functools

import jax
import jax.numpy as jnp
from jax import lax
from jax.experimental import pallas as pl
from jax.experimental.pallas import tpu as pltpu
from jax.experimental.pallas import tpu_sc as plsc

_BATCH = 4096
_PIXELS = 12288


def _sc_gather(idx, a_flat, b_flat):
    info = plsc.get_sparse_core_info()
    nw = info.num_cores * info.num_subcores
    b_per_w = _BATCH // nw

    mesh = plsc.VectorSubcoreMesh(core_axis_name="c", subcore_axis_name="s")

    @functools.partial(
        pl.kernel,
        out_type=(
            jax.ShapeDtypeStruct((_BATCH,), jnp.float32),
            jax.ShapeDtypeStruct((_BATCH,), jnp.float32),
        ),
        mesh=mesh,
        scratch_types=[
            pltpu.VMEM((b_per_w,), jnp.int32),
            pltpu.VMEM((b_per_w,), jnp.float32),
            pltpu.VMEM((b_per_w,), jnp.float32),
            pltpu.SemaphoreType.DMA,
            pltpu.SemaphoreType.DMA,
        ],
    )
    def sc_kernel(idx_hbm, a_hbm, b_hbm, scale_hbm, off_hbm,
                  idx_v, a_v, b_v, sem_a, sem_b):
        wid = lax.axis_index("s") * info.num_cores + lax.axis_index("c")
        base = wid * b_per_w
        pltpu.sync_copy(idx_hbm.at[pl.ds(base, b_per_w)], idx_v)
        cp_a = pltpu.async_copy(a_hbm.at[idx_v], a_v, sem_a)
        cp_b = pltpu.async_copy(b_hbm.at[idx_v], b_v, sem_b)
        cp_a.wait()
        cp_b.wait()
        for i in range(b_per_w // 16):
            sl = pl.ds(i * 16, 16)
            a_v[sl] = jnp.exp(a_v[sl])
        pltpu.sync_copy(a_v, scale_hbm.at[pl.ds(base, b_per_w)])
        pltpu.sync_copy(b_v, off_hbm.at[pl.ds(base, b_per_w)])

    return sc_kernel(idx, a_flat, b_flat)


def _tc_apply(scale, offset, image):
    rows = 256
    grid = (_BATCH // rows,)

    def body(s_ref, o_ref, img_ref, out_ref):
        s = s_ref[...].reshape(rows, 1)
        o = o_ref[...].reshape(rows, 1)
        out_ref[...] = jnp.clip(s * img_ref[...] + o, 0.0, 1.0)

    return pl.pallas_call(
        body,
        grid=grid,
        in_specs=[
            pl.BlockSpec((rows,), lambda i: (i,)),
            pl.BlockSpec((rows,), lambda i: (i,)),
            pl.BlockSpec((rows, _PIXELS), lambda i: (i, 0)),
        ],
        out_specs=pl.BlockSpec((rows, _PIXELS), lambda i: (i, 0)),
        out_shape=jax.ShapeDtypeStruct((_BATCH, _PIXELS), jnp.float32),
        compiler_params=pltpu.CompilerParams(
            vmem_limit_bytes=110 * 1024 * 1024),
    )(scale, offset, image)


def kernel(idx, image, exposure_a, exposure_b):
    scale, offset = _sc_gather(idx, exposure_a.reshape(-1),
                               exposure_b.reshape(-1))
    return _tc_apply(scale, offset, image)
